# baseline (device time: 78797 ns/iter reference)
import functools

import numpy as np
import jax
import jax.numpy as jnp
from jax import lax
from jax.experimental import pallas as pl
from jax.experimental.pallas import tpu as pltpu

N_DEV = 8
N_ROUNDS = 3


def _mask_bias(sq: int, skv: int, block: int) -> np.ndarray:
    qb = (np.arange(sq) // block)[:, None]
    kb = (np.arange(skv) // block)[None, :]
    mask = (qb == kb) | (kb == 0) | ((qb + kb) % 3 == 0)
    return np.where(mask, 0.0, -1e9).astype(np.float32)


def kernel(x, Wq, K_ext, V_ext, Wo):
    B, Sq, E = x.shape
    _, Skv, H, Dh = K_ext.shape
    HD = H * Dh

    my = lax.axis_index("i")
    Wq_i = lax.dynamic_slice(Wq, (0, my * HD), (E, HD))
    Wo_i = lax.dynamic_slice(Wo, (my * HD, 0), (HD, E))
    Kt = K_ext.transpose(0, 2, 1, 3)
    Vt = V_ext.transpose(0, 2, 1, 3)
    bias = jnp.asarray(_mask_bias(Sq, Skv, 64))

    def body(x_ref, wq_ref, kt_ref, vt_ref, wo_ref, bias_ref,
             out_ref, send_ref, recv_ref, send_sems, recv_sems):
        my_pos = lax.axis_index("i")

        barrier = pltpu.get_barrier_semaphore()
        for r in range(N_ROUNDS):
            peer = my_pos ^ (1 << r)
            pl.semaphore_signal(
                barrier, inc=1,
                device_id=(peer,), device_id_type=pl.DeviceIdType.MESH,
            )
        pl.semaphore_wait(barrier, N_ROUNDS)

        wq = wq_ref[...].astype(jnp.bfloat16)
        wo = wo_ref[...].astype(jnp.bfloat16)
        b_mat = bias_ref[...]
        for b in range(B):
            xb = x_ref[b].astype(jnp.bfloat16)
            q_all = jnp.dot(xb, wq, preferred_element_type=jnp.float32)
            ctxs = []
            for h in range(H):
                q = q_all[:, h * Dh:(h + 1) * Dh].astype(jnp.bfloat16)
                k = kt_ref[b, h].astype(jnp.bfloat16)
                s = lax.dot_general(
                    q, k, (((1,), (1,)), ((), ())),
                    preferred_element_type=jnp.float32,
                )
                s = s * 0.125 + b_mat
                m = jnp.max(s, axis=1, keepdims=True)
                p = jnp.exp(s - m)
                p = p / jnp.sum(p, axis=1, keepdims=True)
                v = vt_ref[b, h].astype(jnp.bfloat16)
                ctxs.append(jnp.dot(p.astype(jnp.bfloat16), v,
                                    preferred_element_type=jnp.float32))
            ctx = jnp.concatenate(ctxs, axis=1).astype(jnp.bfloat16)
            out_ref[b] = jnp.dot(ctx, wo, preferred_element_type=jnp.float32)

        for r in range(N_ROUNDS):
            peer = my_pos ^ (1 << r)
            send_ref[r] = out_ref[...].astype(jnp.bfloat16)
            rdma = pltpu.make_async_remote_copy(
                src_ref=send_ref.at[r],
                dst_ref=recv_ref.at[r],
                send_sem=send_sems.at[r],
                recv_sem=recv_sems.at[r],
                device_id=(peer,),
                device_id_type=pl.DeviceIdType.MESH,
            )
            rdma.start()
            rdma.wait()
            out_ref[...] += recv_ref[r].astype(jnp.float32)

    return pl.pallas_call(
        body,
        out_shape=jax.ShapeDtypeStruct((B, Sq, E), jnp.float32),
        in_specs=[pl.BlockSpec(memory_space=pltpu.VMEM)] * 6,
        out_specs=pl.BlockSpec(memory_space=pltpu.VMEM),
        scratch_shapes=[
            pltpu.VMEM((N_ROUNDS, B, Sq, E), jnp.bfloat16),
            pltpu.VMEM((N_ROUNDS, B, Sq, E), jnp.bfloat16),
            pltpu.SemaphoreType.DMA((N_ROUNDS,)),
            pltpu.SemaphoreType.DMA((N_ROUNDS,)),
        ],
        compiler_params=pltpu.CompilerParams(collective_id=0),
    )(x, Wq_i, Kt, Vt, Wo_i, bias)


# device time: 63165 ns/iter; 1.2475x vs baseline; 1.2475x over previous
import functools

import numpy as np
import jax
import jax.numpy as jnp
from jax import lax
from jax.experimental import pallas as pl
from jax.experimental.pallas import tpu as pltpu

N_DEV = 8
N_ROUNDS = 3


def _mask_bias(sq: int, skv: int, block: int) -> np.ndarray:
    qb = (np.arange(sq) // block)[:, None]
    kb = (np.arange(skv) // block)[None, :]
    mask = (qb == kb) | (kb == 0) | ((qb + kb) % 3 == 0)
    return np.where(mask, 0.0, -1e9).astype(np.float32)


def kernel(x, Wq, K_ext, V_ext, Wo):
    B, Sq, E = x.shape
    _, Skv, H, Dh = K_ext.shape
    HD = H * Dh

    my = lax.axis_index("i")
    Wq_i = lax.dynamic_slice(Wq, (0, my * HD), (E, HD))
    Wo_i = lax.dynamic_slice(Wo, (my * HD, 0), (HD, E))
    Kt = K_ext.transpose(0, 2, 1, 3)
    Vt = V_ext.transpose(0, 2, 1, 3)
    bias = jnp.asarray(_mask_bias(Sq, Skv, 64))

    RS_SIZES = (Sq // 2, Sq // 4, Sq // 8)

    def body(x_ref, wq_ref, kt_ref, vt_ref, wo_ref, bias_ref,
             out_ref,
             s0, s1, s2, s3, s4, s5, r0, r1, r2, r3, r4, r5,
             send_sems, recv_sems):
        send_bufs = (s0, s1, s2, s3, s4, s5)
        recv_bufs = (r0, r1, r2, r3, r4, r5)
        my_pos = lax.axis_index("i")

        barrier = pltpu.get_barrier_semaphore()
        for r in range(N_ROUNDS):
            peer = my_pos ^ (1 << r)
            pl.semaphore_signal(
                barrier, inc=1,
                device_id=(peer,), device_id_type=pl.DeviceIdType.MESH,
            )
        pl.semaphore_wait(barrier, N_ROUNDS)

        wq = wq_ref[...].astype(jnp.bfloat16)
        wo = wo_ref[...].astype(jnp.bfloat16)
        b_mat = bias_ref[...]
        for b in range(B):
            xb = x_ref[b].astype(jnp.bfloat16)
            q_all = jnp.dot(xb, wq, preferred_element_type=jnp.float32)
            ctxs = []
            for h in range(H):
                q = q_all[:, h * Dh:(h + 1) * Dh].astype(jnp.bfloat16)
                k = kt_ref[b, h].astype(jnp.bfloat16)
                s = lax.dot_general(
                    q, k, (((1,), (1,)), ((), ())),
                    preferred_element_type=jnp.float32,
                )
                s = s * 0.125 + b_mat
                m = jnp.max(s, axis=1, keepdims=True)
                p = jnp.exp(s - m)
                p = p / jnp.sum(p, axis=1, keepdims=True)
                v = vt_ref[b, h].astype(jnp.bfloat16)
                ctxs.append(jnp.dot(p.astype(jnp.bfloat16), v,
                                    preferred_element_type=jnp.float32))
            ctx = jnp.concatenate(ctxs, axis=1).astype(jnp.bfloat16)
            out_ref[b] = jnp.dot(ctx, wo, preferred_element_type=jnp.float32)

        bits = [(my_pos >> r) & 1 for r in range(N_ROUNDS)]

        def xchg(idx, peer, src_off, dst_buf_len):
            n = dst_buf_len
            send_bufs[idx][...] = out_ref[:, pl.ds(src_off, n), :].astype(
                jnp.bfloat16)
            rdma = pltpu.make_async_remote_copy(
                src_ref=send_bufs[idx],
                dst_ref=recv_bufs[idx],
                send_sem=send_sems.at[idx],
                recv_sem=recv_sems.at[idx],
                device_id=(peer,),
                device_id_type=pl.DeviceIdType.MESH,
            )
            rdma.start()
            rdma.wait()

        base = 0
        bases = []
        for r in range(N_ROUNDS):
            n = RS_SIZES[r]
            bases.append(base)
            keep_off = base + n * bits[r]
            send_off = base + n * (1 - bits[r])
            xchg(r, my_pos ^ (1 << r), send_off, n)
            out_ref[:, pl.ds(keep_off, n), :] += recv_bufs[r][...].astype(
                jnp.float32)
            base = keep_off

        for idx, r in enumerate(reversed(range(N_ROUNDS))):
            n = RS_SIZES[r]
            my_off = bases[r] + n * bits[r]
            sib_off = bases[r] + n * (1 - bits[r])
            xchg(N_ROUNDS + idx, my_pos ^ (1 << r), my_off, n)
            out_ref[:, pl.ds(sib_off, n), :] = recv_bufs[N_ROUNDS + idx][
                ...].astype(jnp.float32)

    return pl.pallas_call(
        body,
        out_shape=jax.ShapeDtypeStruct((B, Sq, E), jnp.float32),
        in_specs=[pl.BlockSpec(memory_space=pltpu.VMEM)] * 6,
        out_specs=pl.BlockSpec(memory_space=pltpu.VMEM),
        scratch_shapes=(
            [pltpu.VMEM((B, n, E), jnp.bfloat16)
             for n in list(RS_SIZES) + list(reversed(RS_SIZES))] * 2
            + [pltpu.SemaphoreType.DMA((2 * N_ROUNDS,)),
               pltpu.SemaphoreType.DMA((2 * N_ROUNDS,))]
        ),
        compiler_params=pltpu.CompilerParams(collective_id=0),
    )(x, Wq_i, Kt, Vt, Wo_i, bias)


# device time: 46528 ns/iter; 1.6935x vs baseline; 1.3576x over previous
import numpy as np
import jax
import jax.numpy as jnp
from jax import lax
from jax.experimental import pallas as pl
from jax.experimental.pallas import tpu as pltpu

N_DEV = 8
N_PEERS = N_DEV - 1


def _mask_bias(sq: int, skv: int, block: int) -> np.ndarray:
    qb = (np.arange(sq) // block)[:, None]
    kb = (np.arange(skv) // block)[None, :]
    mask = (qb == kb) | (kb == 0) | ((qb + kb) % 3 == 0)
    return np.where(mask, 0.0, -1e9).astype(np.float32)


def _chunk_off(pos, rows):
    b0, b1, b2 = pos & 1, (pos >> 1) & 1, (pos >> 2) & 1
    return rows * (4 * b0 + 2 * b1 + b2)


def kernel(x, Wq, K_ext, V_ext, Wo):
    B, Sq, E = x.shape
    _, Skv, H, Dh = K_ext.shape
    HD = H * Dh
    CH = Sq // N_DEV

    my = lax.axis_index("i")
    Wq_i = lax.dynamic_slice(Wq, (0, my * HD), (E, HD))
    Wo_i = lax.dynamic_slice(Wo, (my * HD, 0), (HD, E))
    Kt = K_ext.transpose(0, 2, 1, 3)
    Vt = V_ext.transpose(0, 2, 1, 3)
    bias = jnp.asarray(_mask_bias(Sq, Skv, 64))

    def body(x_ref, wq_ref, kt_ref, vt_ref, wo_ref, bias_ref,
             out_ref, acc_ref, rs_s, rs_r,
             rs_send_sems, rs_recv_sems, ag_send_sems, ag_recv_sems):
        my_pos = lax.axis_index("i")

        barrier = pltpu.get_barrier_semaphore()
        for k in range(1, N_DEV):
            pl.semaphore_signal(
                barrier, inc=1,
                device_id=(my_pos ^ k,), device_id_type=pl.DeviceIdType.MESH,
            )
        pl.semaphore_wait(barrier, N_PEERS)

        wq = wq_ref[...].astype(jnp.bfloat16)
        wo = wo_ref[...].astype(jnp.bfloat16)
        b_mat = bias_ref[...]
        for b in range(B):
            xb = x_ref[b].astype(jnp.bfloat16)
            q_all = jnp.dot(xb, wq, preferred_element_type=jnp.float32)
            ctxs = []
            for h in range(H):
                q = q_all[:, h * Dh:(h + 1) * Dh].astype(jnp.bfloat16)
                k = kt_ref[b, h].astype(jnp.bfloat16)
                s = lax.dot_general(
                    q, k, (((1,), (1,)), ((), ())),
                    preferred_element_type=jnp.float32,
                )
                s = s * 0.125 + b_mat
                m = jnp.max(s, axis=1, keepdims=True)
                p = jnp.exp(s - m)
                p = p / jnp.sum(p, axis=1, keepdims=True)
                v = vt_ref[b, h].astype(jnp.bfloat16)
                ctxs.append(jnp.dot(p.astype(jnp.bfloat16), v,
                                    preferred_element_type=jnp.float32))
            ctx = jnp.concatenate(ctxs, axis=1).astype(jnp.bfloat16)
            acc_ref[b] = jnp.dot(ctx, wo, preferred_element_type=jnp.float32)

        rs = []
        for k in range(1, N_DEV):
            peer = my_pos ^ k
            off = _chunk_off(peer, CH)
            rs_s[k - 1] = acc_ref[:, pl.ds(off, CH), :].astype(jnp.bfloat16)
            rdma = pltpu.make_async_remote_copy(
                src_ref=rs_s.at[k - 1],
                dst_ref=rs_r.at[k - 1],
                send_sem=rs_send_sems.at[k - 1],
                recv_sem=rs_recv_sems.at[k - 1],
                device_id=(peer,),
                device_id_type=pl.DeviceIdType.MESH,
            )
            rdma.start()
            rs.append(rdma)
        for rdma in rs:
            rdma.wait_send()
        for rdma in rs:
            rdma.wait_recv()

        my_off = _chunk_off(my_pos, CH)
        total = acc_ref[:, pl.ds(my_off, CH), :]
        for k in range(1, N_DEV):
            total = total + rs_r[k - 1].astype(jnp.float32)
        out_ref[:, pl.ds(my_off, CH), :] = total.astype(jnp.bfloat16)

        ag = []
        for k in range(1, N_DEV):
            peer = my_pos ^ k
            rdma = pltpu.make_async_remote_copy(
                src_ref=out_ref.at[:, pl.ds(my_off, CH), :],
                dst_ref=out_ref.at[:, pl.ds(my_off, CH), :],
                send_sem=ag_send_sems.at[k - 1],
                recv_sem=ag_recv_sems.at[k - 1],
                device_id=(peer,),
                device_id_type=pl.DeviceIdType.MESH,
            )
            rdma.start()
            ag.append(rdma)
        for rdma in ag:
            rdma.wait_send()
        for rdma in ag:
            rdma.wait_recv()

    return pl.pallas_call(
        body,
        out_shape=jax.ShapeDtypeStruct((B, Sq, E), jnp.bfloat16),
        in_specs=[pl.BlockSpec(memory_space=pltpu.VMEM)] * 6,
        out_specs=pl.BlockSpec(memory_space=pltpu.VMEM),
        scratch_shapes=[
            pltpu.VMEM((B, Sq, E), jnp.float32),
            pltpu.VMEM((N_PEERS, B, CH, E), jnp.bfloat16),
            pltpu.VMEM((N_PEERS, B, CH, E), jnp.bfloat16),
            pltpu.SemaphoreType.DMA((N_PEERS,)),
            pltpu.SemaphoreType.DMA((N_PEERS,)),
            pltpu.SemaphoreType.DMA((N_PEERS,)),
            pltpu.SemaphoreType.DMA((N_PEERS,)),
        ],
        compiler_params=pltpu.CompilerParams(collective_id=0),
    )(x, Wq_i, Kt, Vt, Wo_i, bias)
